# baseline (speedup 1.0000x reference)
import functools

import jax
import jax.numpy as jnp
from jax import lax
from jax.experimental import pallas as pl
from jax.experimental.pallas import tpu as pltpu
from jax.experimental.pallas import tpu_sc as plsc

_TT = 512
_KC = 2736
_NKC = 3
_NSUB = 6
_SUB = 456
_RPS = _SUB // 8


def _argmin_body(cb_ref, z_ref, zn_ref, cn_ref, out_ref, acc_v, acc_i):
    k = pl.program_id(1)
    nk = pl.num_programs(1)
    zb = z_ref[0]
    zn = zn_ref[...]
    run_v = jnp.full((8, _TT), jnp.inf, jnp.float32)
    run_r = jnp.zeros((8, _TT), jnp.int32)
    for sub in range(_NSUB):
        g = lax.dot_general(
            cb_ref[pl.ds(k * _KC + sub * _SUB, _SUB), :], zb,
            (((1,), (0,)), ((), ())),
            preferred_element_type=jnp.float32,
        )
        cn_sub = cn_ref[sub * _SUB:(sub + 1) * _SUB, :]
        for r in range(_RPS):
            row0 = r * 8
            x = zn + g[row0:row0 + 8, :] + cn_sub[row0:row0 + 8, :]
            lt = x < run_v
            run_v = jnp.where(lt, x, run_v)
            run_r = jnp.where(lt, jnp.int32(sub * _RPS + r), run_r)
    srow = lax.broadcasted_iota(jnp.int32, (8, _TT), 0)
    idx = run_r * 8 + srow + k * _KC
    m = jnp.min(run_v, axis=0, keepdims=True)
    a = jnp.min(jnp.where(run_v == m, idx, jnp.int32(2 ** 30)),
                axis=0, keepdims=True)

    @pl.when(k == 0)
    def _():
        acc_v[...] = m.astype(jnp.bfloat16).astype(jnp.float32)
        acc_i[...] = a

    @pl.when(k > 0)
    def _():
        accw = acc_v[...]
        keep = accw <= m
        acc_v[...] = jnp.where(keep, accw, m).astype(
            jnp.bfloat16).astype(jnp.float32)
        acc_i[...] = jnp.where(keep, acc_i[...], a)

    @pl.when(k == nk - 1)
    def _():
        out_ref[...] = acc_i[...]


def _argmin_indices(z, zn, cb_pad, cn):
    b, c_dim, t_len = z.shape
    n_tok = b * t_len
    nt = n_tok // _TT
    per_b = t_len // _TT
    kpad = cb_pad.shape[0]
    idx2d = pl.pallas_call(
        _argmin_body,
        grid=(nt, _NKC),
        in_specs=[
            pl.BlockSpec((kpad, c_dim), lambda t, k: (0, 0)),
            pl.BlockSpec((1, c_dim, _TT),
                         lambda t, k: (t // per_b, 0, t % per_b)),
            pl.BlockSpec((1, _TT), lambda t, k: (0, t)),
            pl.BlockSpec((_KC, 1), lambda t, k: (k, 0)),
        ],
        out_specs=pl.BlockSpec((1, _TT), lambda t, k: (0, t)),
        out_shape=jax.ShapeDtypeStruct((1, n_tok), jnp.int32),
        scratch_shapes=[
            pltpu.VMEM((1, _TT), jnp.float32),
            pltpu.VMEM((1, _TT), jnp.int32),
        ],
        compiler_params=pltpu.CompilerParams(
            dimension_semantics=("parallel", "arbitrary")),
    )(cb_pad, z, zn, cn)
    return idx2d.reshape(n_tok)


def _sc_gather(table, idx):
    n_tok = idx.shape[0]
    d = table.shape[1]
    info = plsc.get_sparse_core_info()
    nw = info.num_cores * info.num_subcores
    b_per_w = n_tok // nw
    chunk = 128
    n_chunks = b_per_w // chunk
    mesh = plsc.VectorSubcoreMesh(core_axis_name="c", subcore_axis_name="s")

    @functools.partial(
        pl.kernel, mesh=mesh,
        out_type=jax.ShapeDtypeStruct((n_tok, d), jnp.float32),
        scratch_types=[
            pltpu.VMEM((chunk,), jnp.int32),
            pltpu.VMEM((chunk, d), jnp.float32),
            pltpu.SemaphoreType.DMA,
        ],
    )
    def gather_kernel(table_hbm, idx_hbm, out_hbm, idx_v, rows_v, sem):
        wid = lax.axis_index("s") * info.num_cores + lax.axis_index("c")
        base = wid * b_per_w
        for ci in range(n_chunks):
            off = base + ci * chunk
            pltpu.sync_copy(idx_hbm.at[pl.ds(off, chunk)], idx_v)
            pltpu.async_copy(table_hbm.at[idx_v], rows_v, sem).wait()
            pltpu.sync_copy(rows_v, out_hbm.at[pl.ds(off, chunk)])

    return gather_kernel(table, idx)


def kernel(z, codebook):
    b, c, t = z.shape
    n_codes = codebook.shape[0]
    n_pad = _NKC * _KC - n_codes
    cb_pad = jnp.concatenate(
        [codebook, jnp.full((n_pad, c), 1000.0, dtype=codebook.dtype)], axis=0)
    zn = jnp.sum(jnp.transpose(z, (0, 2, 1)) ** 2, axis=2).reshape(1, -1)
    cn = jnp.sum(cb_pad ** 2, axis=1, keepdims=True)
    z2 = (-2.0 * z).astype(jnp.bfloat16)
    cb16 = cb_pad.astype(jnp.bfloat16)
    indices = _argmin_indices(z2, zn, cb16, cn)
    z_q_flat = _sc_gather(codebook, indices)
    z_q = jnp.transpose(z_q_flat.reshape(b, t, c), (0, 2, 1))
    return (z_q, indices.reshape(b, t))


# speedup vs baseline: 1.0077x; 1.0077x over previous
import functools

import jax
import jax.numpy as jnp
from jax import lax
from jax.experimental import pallas as pl
from jax.experimental.pallas import tpu as pltpu
from jax.experimental.pallas import tpu_sc as plsc

_TT = 512
_KC = 2736
_NKC = 3
_NSUB = 6
_SUB = 456
_RPS = _SUB // 8


def _argmin_body(cb_ref, z_ref, zn_ref, cn_ref, out_ref, acc_v, acc_i):
    k = pl.program_id(1)
    nk = pl.num_programs(1)
    zb = -2.0 * z_ref[0]
    zn = zn_ref[...]
    run_v = jnp.full((8, _TT), jnp.inf, jnp.float32)
    run_r = jnp.zeros((8, _TT), jnp.int32)
    for sub in range(_NSUB):
        g = lax.dot_general(
            cb_ref[pl.ds(k * _KC + sub * _SUB, _SUB), :], zb,
            (((1,), (0,)), ((), ())),
            preferred_element_type=jnp.float32,
        )
        cn_sub = cn_ref[sub * _SUB:(sub + 1) * _SUB, :]
        for r in range(_RPS):
            row0 = r * 8
            x = zn + g[row0:row0 + 8, :] + cn_sub[row0:row0 + 8, :]
            lt = x < run_v
            run_v = jnp.where(lt, x, run_v)
            run_r = jnp.where(lt, jnp.int32(sub * _RPS + r), run_r)
    srow = lax.broadcasted_iota(jnp.int32, (8, _TT), 0)
    idx = run_r * 8 + srow + k * _KC
    m = jnp.min(run_v, axis=0, keepdims=True)
    a = jnp.min(jnp.where(run_v == m, idx, jnp.int32(2 ** 30)),
                axis=0, keepdims=True)

    @pl.when(k == 0)
    def _():
        acc_v[...] = m.astype(jnp.bfloat16).astype(jnp.float32)
        acc_i[...] = a

    @pl.when(k > 0)
    def _():
        accw = acc_v[...]
        keep = accw <= m
        acc_v[...] = jnp.where(keep, accw, m).astype(
            jnp.bfloat16).astype(jnp.float32)
        acc_i[...] = jnp.where(keep, acc_i[...], a)

    @pl.when(k == nk - 1)
    def _():
        out_ref[...] = acc_i[...]


def _argmin_indices(z, zn, cb_pad, cn):
    b, c_dim, t_len = z.shape
    n_tok = b * t_len
    nt = n_tok // _TT
    per_b = t_len // _TT
    kpad = cb_pad.shape[0]
    idx2d = pl.pallas_call(
        _argmin_body,
        grid=(nt, _NKC),
        in_specs=[
            pl.BlockSpec((kpad, c_dim), lambda t, k: (0, 0)),
            pl.BlockSpec((1, c_dim, _TT),
                         lambda t, k: (t // per_b, 0, t % per_b)),
            pl.BlockSpec((1, _TT), lambda t, k: (0, t)),
            pl.BlockSpec((_KC, 1), lambda t, k: (k, 0)),
        ],
        out_specs=pl.BlockSpec((1, _TT), lambda t, k: (0, t)),
        out_shape=jax.ShapeDtypeStruct((1, n_tok), jnp.int32),
        scratch_shapes=[
            pltpu.VMEM((1, _TT), jnp.float32),
            pltpu.VMEM((1, _TT), jnp.int32),
        ],
        compiler_params=pltpu.CompilerParams(
            dimension_semantics=("parallel", "arbitrary")),
    )(cb_pad, z, zn, cn)
    return idx2d.reshape(n_tok)


def _sc_gather(table, idx):
    n_tok = idx.shape[0]
    d = table.shape[1]
    info = plsc.get_sparse_core_info()
    nw = info.num_cores * info.num_subcores
    b_per_w = n_tok // nw
    chunk = 128
    n_chunks = b_per_w // chunk
    mesh = plsc.VectorSubcoreMesh(core_axis_name="c", subcore_axis_name="s")

    @functools.partial(
        pl.kernel, mesh=mesh,
        out_type=jax.ShapeDtypeStruct((n_tok, d), jnp.float32),
        scratch_types=[
            pltpu.VMEM((chunk,), jnp.int32),
            pltpu.VMEM((chunk,), jnp.int32),
            pltpu.VMEM((chunk, d), jnp.float32),
            pltpu.VMEM((chunk, d), jnp.float32),
            pltpu.SemaphoreType.DMA,
            pltpu.SemaphoreType.DMA,
            pltpu.SemaphoreType.DMA,
            pltpu.SemaphoreType.DMA,
        ],
    )
    def gather_kernel(table_hbm, idx_hbm, out_hbm,
                      idx0, idx1, rows0, rows1, g0, g1, s0, s1):
        wid = lax.axis_index("s") * info.num_cores + lax.axis_index("c")
        base = wid * b_per_w
        idxs, rows, gsem, ssem = (idx0, idx1), (rows0, rows1), (g0, g1), (s0, s1)
        gd = [None, None]
        sd = [None, None]
        for ci in range(n_chunks + 1):
            b = ci % 2
            if ci < n_chunks:
                if sd[b] is not None:
                    sd[b].wait()
                off = base + ci * chunk
                pltpu.sync_copy(idx_hbm.at[pl.ds(off, chunk)], idxs[b])
                gd[b] = pltpu.async_copy(table_hbm.at[idxs[b]], rows[b],
                                         gsem[b])
            if ci >= 1:
                pb = (ci - 1) % 2
                gd[pb].wait()
                poff = base + (ci - 1) * chunk
                sd[pb] = pltpu.async_copy(rows[pb],
                                          out_hbm.at[pl.ds(poff, chunk)],
                                          ssem[pb])
        sd[(n_chunks - 2) % 2].wait()
        sd[(n_chunks - 1) % 2].wait()

    return gather_kernel(table, idx)


def kernel(z, codebook):
    b, c, t = z.shape
    n_codes = codebook.shape[0]
    n_pad = _NKC * _KC - n_codes
    cb_pad = jnp.concatenate(
        [codebook, jnp.full((n_pad, c), 1000.0, dtype=codebook.dtype)], axis=0)
    zn = jnp.sum(jnp.transpose(z, (0, 2, 1)) ** 2, axis=2).reshape(1, -1)
    cn = jnp.sum(cb_pad ** 2, axis=1, keepdims=True)
    indices = _argmin_indices(z, zn, cb_pad, cn)
    z_q_flat = _sc_gather(codebook, indices)
    z_q = jnp.transpose(z_q_flat.reshape(b, t, c), (0, 2, 1))
    return (z_q, indices.reshape(b, t))


# speedup vs baseline: 1.1315x; 1.1228x over previous
import functools

import jax
import jax.numpy as jnp
from jax import lax
from jax.experimental import pallas as pl
from jax.experimental.pallas import tpu as pltpu
from jax.experimental.pallas import tpu_sc as plsc

_TT = 1024
_KC = 2736
_NKC = 3
_NSUB = 6
_SUB = 456
_RPS = _SUB // 8


def _argmin_body(cb_ref, z_ref, zn_ref, cn_ref, out_ref, acc_v, acc_i):
    k = pl.program_id(1)
    nk = pl.num_programs(1)
    zb = -2.0 * z_ref[0]
    zn = zn_ref[...]
    run_v = jnp.full((8, _TT), jnp.inf, jnp.float32)
    run_r = jnp.zeros((8, _TT), jnp.int32)
    for sub in range(_NSUB):
        g = lax.dot_general(
            cb_ref[pl.ds(k * _KC + sub * _SUB, _SUB), :], zb,
            (((1,), (0,)), ((), ())),
            preferred_element_type=jnp.float32,
        )
        cn_sub = cn_ref[sub * _SUB:(sub + 1) * _SUB, :]
        for r in range(_RPS):
            row0 = r * 8
            x = zn + g[row0:row0 + 8, :] + cn_sub[row0:row0 + 8, :]
            lt = x < run_v
            run_v = jnp.where(lt, x, run_v)
            run_r = jnp.where(lt, jnp.int32(sub * _RPS + r), run_r)
    srow = lax.broadcasted_iota(jnp.int32, (8, _TT), 0)
    idx = run_r * 8 + srow + k * _KC
    m = jnp.min(run_v, axis=0, keepdims=True)
    a = jnp.min(jnp.where(run_v == m, idx, jnp.int32(2 ** 30)),
                axis=0, keepdims=True)

    @pl.when(k == 0)
    def _():
        acc_v[...] = m.astype(jnp.bfloat16).astype(jnp.float32)
        acc_i[...] = a

    @pl.when(k > 0)
    def _():
        accw = acc_v[...]
        keep = accw <= m
        acc_v[...] = jnp.where(keep, accw, m).astype(
            jnp.bfloat16).astype(jnp.float32)
        acc_i[...] = jnp.where(keep, acc_i[...], a)

    @pl.when(k == nk - 1)
    def _():
        out_ref[...] = acc_i[...]


def _argmin_indices(z, zn, cb_pad, cn):
    b, c_dim, t_len = z.shape
    n_tok = b * t_len
    nt = n_tok // _TT
    per_b = t_len // _TT
    kpad = cb_pad.shape[0]
    idx2d = pl.pallas_call(
        _argmin_body,
        grid=(nt, _NKC),
        in_specs=[
            pl.BlockSpec((kpad, c_dim), lambda t, k: (0, 0)),
            pl.BlockSpec((1, c_dim, _TT),
                         lambda t, k: (t // per_b, 0, t % per_b)),
            pl.BlockSpec((1, _TT), lambda t, k: (0, t)),
            pl.BlockSpec((_KC, 1), lambda t, k: (k, 0)),
        ],
        out_specs=pl.BlockSpec((1, _TT), lambda t, k: (0, t)),
        out_shape=jax.ShapeDtypeStruct((1, n_tok), jnp.int32),
        scratch_shapes=[
            pltpu.VMEM((1, _TT), jnp.float32),
            pltpu.VMEM((1, _TT), jnp.int32),
        ],
        compiler_params=pltpu.CompilerParams(
            dimension_semantics=("parallel", "arbitrary")),
    )(cb_pad, z, zn, cn)
    return idx2d.reshape(n_tok)


def _sc_gather(table, idx):
    n_tok = idx.shape[0]
    d = table.shape[1]
    info = plsc.get_sparse_core_info()
    nw = info.num_cores * info.num_subcores
    b_per_w = n_tok // nw
    chunk = 128
    n_chunks = b_per_w // chunk
    mesh = plsc.VectorSubcoreMesh(core_axis_name="c", subcore_axis_name="s")

    @functools.partial(
        pl.kernel, mesh=mesh,
        out_type=jax.ShapeDtypeStruct((n_tok, d), jnp.float32),
        scratch_types=[
            pltpu.VMEM((chunk,), jnp.int32),
            pltpu.VMEM((chunk,), jnp.int32),
            pltpu.VMEM((chunk, d), jnp.float32),
            pltpu.VMEM((chunk, d), jnp.float32),
            pltpu.SemaphoreType.DMA,
            pltpu.SemaphoreType.DMA,
            pltpu.SemaphoreType.DMA,
            pltpu.SemaphoreType.DMA,
        ],
    )
    def gather_kernel(table_hbm, idx_hbm, out_hbm,
                      idx0, idx1, rows0, rows1, g0, g1, s0, s1):
        wid = lax.axis_index("s") * info.num_cores + lax.axis_index("c")
        base = wid * b_per_w
        idxs, rows, gsem, ssem = (idx0, idx1), (rows0, rows1), (g0, g1), (s0, s1)
        gd = [None, None]
        sd = [None, None]
        for ci in range(n_chunks + 1):
            b = ci % 2
            if ci < n_chunks:
                if sd[b] is not None:
                    sd[b].wait()
                off = base + ci * chunk
                pltpu.sync_copy(idx_hbm.at[pl.ds(off, chunk)], idxs[b])
                gd[b] = pltpu.async_copy(table_hbm.at[idxs[b]], rows[b],
                                         gsem[b])
            if ci >= 1:
                pb = (ci - 1) % 2
                gd[pb].wait()
                poff = base + (ci - 1) * chunk
                sd[pb] = pltpu.async_copy(rows[pb],
                                          out_hbm.at[pl.ds(poff, chunk)],
                                          ssem[pb])
        sd[(n_chunks - 2) % 2].wait()
        sd[(n_chunks - 1) % 2].wait()

    return gather_kernel(table, idx)


def kernel(z, codebook):
    b, c, t = z.shape
    n_codes = codebook.shape[0]
    n_pad = _NKC * _KC - n_codes
    cb_pad = jnp.concatenate(
        [codebook, jnp.full((n_pad, c), 1000.0, dtype=codebook.dtype)], axis=0)
    zn = jnp.sum(jnp.transpose(z, (0, 2, 1)) ** 2, axis=2).reshape(1, -1)
    cn = jnp.sum(cb_pad ** 2, axis=1, keepdims=True)
    indices = _argmin_indices(z, zn, cb_pad, cn)
    z_q_flat = _sc_gather(codebook, indices)
    z_q = jnp.transpose(z_q_flat.reshape(b, t, c), (0, 2, 1))
    return (z_q, indices.reshape(b, t))


# speedup vs baseline: 1.1561x; 1.0218x over previous
import functools

import jax
import jax.numpy as jnp
from jax import lax
from jax.experimental import pallas as pl
from jax.experimental.pallas import tpu as pltpu
from jax.experimental.pallas import tpu_sc as plsc

_TT = 1024
_KC = 2736
_NKC = 3
_NSUB = 6
_SUB = 456
_RPS = _SUB // 8


def _argmin_body(cb_ref, z_ref, zn_ref, cn_ref, out_ref, acc_v, acc_i):
    k = pl.program_id(1)
    nk = pl.num_programs(1)
    start = k * _KC - 16 * (k // 2)
    last = k == nk - 1
    zb = -2.0 * z_ref[0]
    zn = zn_ref[...]
    run_v = jnp.full((8, _TT), jnp.inf, jnp.float32)
    run_r = jnp.zeros((8, _TT), jnp.int32)
    for sub in range(_NSUB):
        g = lax.dot_general(
            cb_ref[pl.ds(start + sub * _SUB, _SUB), :], zb,
            (((1,), (0,)), ((), ())),
            preferred_element_type=jnp.float32,
        )
        cn_sub = cn_ref[pl.ds(start + sub * _SUB, _SUB), :]
        for r in range(_RPS):
            row0 = r * 8
            x = zn + g[row0:row0 + 8, :] + cn_sub[row0:row0 + 8, :]
            if sub == 0 and r < 2:
                x = jnp.where(last, jnp.inf, x)
            lt = x < run_v
            run_v = jnp.where(lt, x, run_v)
            run_r = jnp.where(lt, jnp.int32(sub * _RPS + r), run_r)
    srow = lax.broadcasted_iota(jnp.int32, (8, _TT), 0)
    idx = run_r * 8 + srow + start
    m = jnp.min(run_v, axis=0, keepdims=True)
    a = jnp.min(jnp.where(run_v == m, idx, jnp.int32(2 ** 30)),
                axis=0, keepdims=True)

    @pl.when(k == 0)
    def _():
        acc_v[...] = m.astype(jnp.bfloat16).astype(jnp.float32)
        acc_i[...] = a

    @pl.when(k > 0)
    def _():
        accw = acc_v[...]
        keep = accw <= m
        acc_v[...] = jnp.where(keep, accw, m).astype(
            jnp.bfloat16).astype(jnp.float32)
        acc_i[...] = jnp.where(keep, acc_i[...], a)

    @pl.when(k == nk - 1)
    def _():
        out_ref[...] = acc_i[...]


def _argmin_indices(z, zn, cb_pad, cn):
    b, c_dim, t_len = z.shape
    n_tok = b * t_len
    nt = n_tok // _TT
    per_b = t_len // _TT
    kpad = cb_pad.shape[0]
    idx2d = pl.pallas_call(
        _argmin_body,
        grid=(nt, _NKC),
        in_specs=[
            pl.BlockSpec((kpad, c_dim), lambda t, k: (0, 0)),
            pl.BlockSpec((1, c_dim, _TT),
                         lambda t, k: (t // per_b, 0, t % per_b)),
            pl.BlockSpec((1, _TT), lambda t, k: (0, t)),
            pl.BlockSpec((kpad, 1), lambda t, k: (0, 0)),
        ],
        out_specs=pl.BlockSpec((1, _TT), lambda t, k: (0, t)),
        out_shape=jax.ShapeDtypeStruct((1, n_tok), jnp.int32),
        scratch_shapes=[
            pltpu.VMEM((1, _TT), jnp.float32),
            pltpu.VMEM((1, _TT), jnp.int32),
        ],
        compiler_params=pltpu.CompilerParams(
            dimension_semantics=("parallel", "arbitrary")),
    )(cb_pad, z, zn, cn)
    return idx2d.reshape(n_tok)


def _sc_gather(table, idx):
    n_tok = idx.shape[0]
    d = table.shape[1]
    info = plsc.get_sparse_core_info()
    nw = info.num_cores * info.num_subcores
    b_per_w = n_tok // nw
    chunk = 128
    n_chunks = b_per_w // chunk
    mesh = plsc.VectorSubcoreMesh(core_axis_name="c", subcore_axis_name="s")

    @functools.partial(
        pl.kernel, mesh=mesh,
        out_type=jax.ShapeDtypeStruct((n_tok, d), jnp.float32),
        scratch_types=[
            pltpu.VMEM((chunk,), jnp.int32),
            pltpu.VMEM((chunk,), jnp.int32),
            pltpu.VMEM((chunk, d), jnp.float32),
            pltpu.VMEM((chunk, d), jnp.float32),
            pltpu.SemaphoreType.DMA,
            pltpu.SemaphoreType.DMA,
            pltpu.SemaphoreType.DMA,
            pltpu.SemaphoreType.DMA,
        ],
    )
    def gather_kernel(table_hbm, idx_hbm, out_hbm,
                      idx0, idx1, rows0, rows1, g0, g1, s0, s1):
        wid = lax.axis_index("s") * info.num_cores + lax.axis_index("c")
        base = wid * b_per_w
        idxs, rows, gsem, ssem = (idx0, idx1), (rows0, rows1), (g0, g1), (s0, s1)
        gd = [None, None]
        sd = [None, None]
        for ci in range(n_chunks + 1):
            b = ci % 2
            if ci < n_chunks:
                if sd[b] is not None:
                    sd[b].wait()
                off = base + ci * chunk
                pltpu.sync_copy(idx_hbm.at[pl.ds(off, chunk)], idxs[b])
                gd[b] = pltpu.async_copy(table_hbm.at[idxs[b]], rows[b],
                                         gsem[b])
            if ci >= 1:
                pb = (ci - 1) % 2
                gd[pb].wait()
                poff = base + (ci - 1) * chunk
                sd[pb] = pltpu.async_copy(rows[pb],
                                          out_hbm.at[pl.ds(poff, chunk)],
                                          ssem[pb])
        sd[(n_chunks - 2) % 2].wait()
        sd[(n_chunks - 1) % 2].wait()

    return gather_kernel(table, idx)


def kernel(z, codebook):
    b, c, t = z.shape
    zn = jnp.sum(jnp.transpose(z, (0, 2, 1)) ** 2, axis=2).reshape(1, -1)
    cn = jnp.sum(codebook ** 2, axis=1, keepdims=True)
    indices = _argmin_indices(z, zn, codebook, cn)
    z_q_flat = _sc_gather(codebook, indices)
    z_q = jnp.transpose(z_q_flat.reshape(b, t, c), (0, 2, 1))
    return (z_q, indices.reshape(b, t))
